# baseline (speedup 1.0000x reference)
import functools

import jax
import jax.numpy as jnp
from jax import lax
from jax.experimental import pallas as pl
from jax.experimental.pallas import tpu as pltpu
from jax.experimental.pallas import tpu_sc as plsc

BATCH = 16384
DIM = 32

_info = plsc.get_sparse_core_info()
_NC, _NS, _L = _info.num_cores, _info.num_subcores, _info.num_lanes
_NW = _NC * _NS
_BPW = BATCH // _NW
_CHUNKS = _BPW // _L


def _two_tower_body(user_ids_hbm, item_ids_hbm, user_emb_hbm, item_emb_hbm,
                    out_hbm, uidx_v, iidx_v, urows_v, irows_v, out_v,
                    sem_u, sem_i):
    wid = lax.axis_index("s") * _NC + lax.axis_index("c")
    base = wid * _BPW

    pltpu.sync_copy(user_ids_hbm.at[pl.ds(base, _BPW)], uidx_v)
    pltpu.sync_copy(item_ids_hbm.at[pl.ds(base, _BPW)], iidx_v)

    u_cp = pltpu.async_copy(user_emb_hbm.at[uidx_v], urows_v, sem_u)
    i_cp = pltpu.async_copy(item_emb_hbm.at[iidx_v], irows_v, sem_i)
    u_cp.wait()
    i_cp.wait()

    lane = lax.iota(jnp.int32, _L)

    def chunk_body(chunk, carry):
        rows = chunk * _L + lane
        acc = jnp.zeros((_L,), jnp.float32)
        for c in range(DIM):
            col = jnp.full((_L,), c, jnp.int32)
            uv = plsc.load_gather(urows_v, [rows, col])
            iv = plsc.load_gather(irows_v, [rows, col])
            acc = acc + uv * iv
        out_v[pl.ds(chunk * _L, _L)] = acc
        return carry

    lax.fori_loop(0, _CHUNKS, chunk_body, 0)

    pltpu.sync_copy(out_v, out_hbm.at[pl.ds(base, _BPW)])


@functools.partial(jax.jit, donate_argnums=())
def _two_tower(user_ids, item_ids, user_emb, item_emb):
    mesh = plsc.VectorSubcoreMesh(core_axis_name="c", subcore_axis_name="s")
    kern = pl.kernel(
        _two_tower_body,
        mesh=mesh,
        compiler_params=pltpu.CompilerParams(
            needs_layout_passes=False, use_tc_tiling_on_sc=False),
        out_type=jax.ShapeDtypeStruct((BATCH,), jnp.float32),
        scratch_types=[
            pltpu.VMEM((_BPW,), jnp.int32),
            pltpu.VMEM((_BPW,), jnp.int32),
            pltpu.VMEM((_BPW, DIM), jnp.float32),
            pltpu.VMEM((_BPW, DIM), jnp.float32),
            pltpu.VMEM((_BPW,), jnp.float32),
            pltpu.SemaphoreType.DMA,
            pltpu.SemaphoreType.DMA,
        ],
    )
    return kern(user_ids, item_ids, user_emb, item_emb)


def kernel(user_ids, item_ids, user_emb, item_emb):
    return _two_tower(user_ids.astype(jnp.int32), item_ids.astype(jnp.int32),
                      user_emb, item_emb)


# speedup vs baseline: 1.0061x; 1.0061x over previous
import jax
import jax.numpy as jnp
from jax import lax
from jax.experimental import pallas as pl
from jax.experimental.pallas import tpu as pltpu
from jax.experimental.pallas import tpu_sc as plsc

BATCH = 16384
DIM = 32

_info = plsc.get_sparse_core_info()
_NC, _NS, _L = _info.num_cores, _info.num_subcores, _info.num_lanes
_NW = _NC * _NS
_BPW = BATCH // _NW
_CHUNKS = _BPW // _L
_NROWS = 1000001 * DIM // _L


def _two_tower_body(user_ids_hbm, item_ids_hbm, user_emb_hbm, item_emb_hbm,
                    out_hbm, uidx_v, iidx_v, urow_idx, irow_idx,
                    urows_v, irows_v, out_v, sem_u, sem_i):
    wid = lax.axis_index("s") * _NC + lax.axis_index("c")
    base = wid * _BPW

    pltpu.sync_copy(user_ids_hbm.at[pl.ds(base, _BPW)], uidx_v)
    pltpu.sync_copy(item_ids_hbm.at[pl.ds(base, _BPW)], iidx_v)

    def idx_body(c, carry):
        sl = pl.ds(c * _L, _L)
        sl_hi = pl.ds(_BPW + c * _L, _L)
        ju = uidx_v[sl] * 2
        ji = iidx_v[sl] * 2
        urow_idx[sl] = ju
        urow_idx[sl_hi] = ju + 1
        irow_idx[sl] = ji
        irow_idx[sl_hi] = ji + 1
        return carry

    lax.fori_loop(0, _CHUNKS, idx_body, 0)

    u_cp = pltpu.async_copy(user_emb_hbm.at[urow_idx], urows_v, sem_u)
    i_cp = pltpu.async_copy(item_emb_hbm.at[irow_idx], irows_v, sem_i)
    u_cp.wait()
    i_cp.wait()

    lane = lax.iota(jnp.int32, _L)

    def chunk_body(chunk, carry):
        rows_lo = chunk * _L + lane
        rows_hi = rows_lo + _BPW
        acc = jnp.zeros((_L,), jnp.float32)
        for c in range(_L):
            col = jnp.full((_L,), c, jnp.int32)
            acc = acc + (plsc.load_gather(urows_v, [rows_lo, col])
                         * plsc.load_gather(irows_v, [rows_lo, col]))
            acc = acc + (plsc.load_gather(urows_v, [rows_hi, col])
                         * plsc.load_gather(irows_v, [rows_hi, col]))
        out_v[pl.ds(chunk * _L, _L)] = acc
        return carry

    lax.fori_loop(0, _CHUNKS, chunk_body, 0)

    pltpu.sync_copy(out_v, out_hbm.at[pl.ds(base, _BPW)])


@jax.jit
def _two_tower(user_ids, item_ids, user_emb_flat, item_emb_flat):
    mesh = plsc.VectorSubcoreMesh(core_axis_name="c", subcore_axis_name="s")
    kern = pl.kernel(
        _two_tower_body,
        mesh=mesh,
        compiler_params=pltpu.CompilerParams(
            needs_layout_passes=False, use_tc_tiling_on_sc=False),
        out_type=jax.ShapeDtypeStruct((BATCH,), jnp.float32),
        scratch_types=[
            pltpu.VMEM((_BPW,), jnp.int32),
            pltpu.VMEM((_BPW,), jnp.int32),
            pltpu.VMEM((2 * _BPW,), jnp.int32),
            pltpu.VMEM((2 * _BPW,), jnp.int32),
            pltpu.VMEM((2 * _BPW, _L), jnp.float32),
            pltpu.VMEM((2 * _BPW, _L), jnp.float32),
            pltpu.VMEM((_BPW,), jnp.float32),
            pltpu.SemaphoreType.DMA,
            pltpu.SemaphoreType.DMA,
        ],
    )
    return kern(user_ids, item_ids, user_emb_flat, item_emb_flat)


def kernel(user_ids, item_ids, user_emb, item_emb):
    return _two_tower(user_ids.astype(jnp.int32), item_ids.astype(jnp.int32),
                      user_emb.reshape(_NROWS, _L),
                      item_emb.reshape(_NROWS, _L))


# speedup vs baseline: 4.0241x; 3.9999x over previous
import jax
import jax.numpy as jnp
from jax import lax
from jax.experimental import pallas as pl
from jax.experimental.pallas import tpu as pltpu
from jax.experimental.pallas import tpu_sc as plsc

BATCH = 16384
DIM = 32

_info = plsc.get_sparse_core_info()
_NC, _NS, _L = _info.num_cores, _info.num_subcores, _info.num_lanes
_NW = _NC * _NS
_BPW = BATCH // _NW
_CHUNKS = _BPW // _L
_K = 8
_LOOK = 4


def _body(user_ids_hbm, item_ids_hbm, ut_hbm, it_hbm, out_hbm,
          uidx_v, iidx_v, tu_v, wu_v, ti_v, wi_v,
          u_ring, i_ring, out_v, sem_u, sem_i):
    wid = lax.axis_index("s") * _NC + lax.axis_index("c")
    base = wid * _BPW

    pltpu.sync_copy(user_ids_hbm.at[pl.ds(base, _BPW)], uidx_v)
    pltpu.sync_copy(item_ids_hbm.at[pl.ds(base, _BPW)], iidx_v)

    def pre_body(c, carry):
        sl = pl.ds(c * _L, _L)
        ju = uidx_v[sl]
        ji = iidx_v[sl]
        tu_v[sl] = lax.shift_right_logical(ju, 7)
        wu_v[sl] = lax.bitwise_and(ju, 127)
        ti_v[sl] = lax.shift_right_logical(ji, 7)
        wi_v[sl] = lax.bitwise_and(ji, 127)
        return carry

    lax.fori_loop(0, _CHUNKS, pre_body, 0)

    lane = lax.iota(jnp.int32, _L)
    lane_hi = lane + _L

    def fire(tu, ti, slot):
        pltpu.async_copy(
            ut_hbm.at[:, pl.ds(pl.multiple_of(tu * 128, 128), 128)],
            u_ring.at[:, pl.ds(slot * 128, 128)], sem_u.at[slot])
        pltpu.async_copy(
            it_hbm.at[:, pl.ds(pl.multiple_of(ti * 128, 128), 128)],
            i_ring.at[:, pl.ds(slot * 128, 128)], sem_i.at[slot])

    def drain(slot):
        pltpu.make_async_copy(
            ut_hbm.at[:, pl.ds(0, 128)],
            u_ring.at[:, pl.ds(slot * 128, 128)], sem_u.at[slot]).wait()
        pltpu.make_async_copy(
            it_hbm.at[:, pl.ds(0, 128)],
            i_ring.at[:, pl.ds(slot * 128, 128)], sem_i.at[slot]).wait()

    def extract(wu, wi, slot):
        cu = jnp.full((_L,), slot * 128 + wu, jnp.int32)
        ci = jnp.full((_L,), slot * 128 + wi, jnp.int32)
        u_lo = plsc.load_gather(u_ring, [lane, cu])
        u_hi = plsc.load_gather(u_ring, [lane_hi, cu])
        i_lo = plsc.load_gather(i_ring, [lane, ci])
        i_hi = plsc.load_gather(i_ring, [lane_hi, ci])
        return jnp.sum(u_lo * i_lo + u_hi * i_hi)

    tu0 = tu_v[pl.ds(0, _L)]
    ti0 = ti_v[pl.ds(0, _L)]
    for l in range(_LOOK):
        fire(tu0[l], ti0[l], l % _K)

    def chunk_body(c, carry):
        sl = pl.ds(c * _L, _L)
        wu_cur = wu_v[sl]
        wi_cur = wi_v[sl]
        tu_cur = tu_v[sl]
        ti_cur = ti_v[sl]
        nxt = pl.ds(jnp.minimum(c + 1, _CHUNKS - 1) * _L, _L)
        tu_nxt = tu_v[nxt]
        ti_nxt = ti_v[nxt]
        acc = jnp.zeros((_L,), jnp.float32)
        for l in range(_L):
            lf = l + _LOOK
            if lf < _L:
                fire(tu_cur[lf], ti_cur[lf], lf % _K)
            else:
                tun = tu_nxt[lf - _L]
                tin = ti_nxt[lf - _L]
                pl.when(c < _CHUNKS - 1)(
                    lambda tun=tun, tin=tin, lf=lf: fire(tun, tin, lf % _K))
            slot = l % _K
            drain(slot)
            s = extract(wu_cur[l], wi_cur[l], slot)
            acc = jnp.where(lane == l, s, acc)
        out_v[sl] = acc
        return carry

    lax.fori_loop(0, _CHUNKS, chunk_body, 0)

    pltpu.sync_copy(out_v, out_hbm.at[pl.ds(base, _BPW)])


@jax.jit
def _two_tower(user_ids, item_ids, user_emb_t, item_emb_t):
    mesh = plsc.VectorSubcoreMesh(core_axis_name="c", subcore_axis_name="s")
    kern = pl.kernel(
        _body,
        mesh=mesh,
        compiler_params=pltpu.CompilerParams(
            needs_layout_passes=False, use_tc_tiling_on_sc=True),
        out_type=jax.ShapeDtypeStruct((BATCH,), jnp.float32),
        scratch_types=[
            pltpu.VMEM((_BPW,), jnp.int32),
            pltpu.VMEM((_BPW,), jnp.int32),
            pltpu.VMEM((_BPW,), jnp.int32),
            pltpu.VMEM((_BPW,), jnp.int32),
            pltpu.VMEM((_BPW,), jnp.int32),
            pltpu.VMEM((_BPW,), jnp.int32),
            pltpu.VMEM((DIM, _K * 128), jnp.float32),
            pltpu.VMEM((DIM, _K * 128), jnp.float32),
            pltpu.VMEM((_BPW,), jnp.float32),
            pltpu.SemaphoreType.DMA((_K,)),
            pltpu.SemaphoreType.DMA((_K,)),
        ],
    )
    return kern(user_ids, item_ids, user_emb_t, item_emb_t)


def kernel(user_ids, item_ids, user_emb, item_emb):
    return _two_tower(user_ids.astype(jnp.int32), item_ids.astype(jnp.int32),
                      user_emb.T, item_emb.T)


# speedup vs baseline: 4.0325x; 1.0021x over previous
import jax
import jax.numpy as jnp
from jax import lax
from jax.experimental import pallas as pl
from jax.experimental.pallas import tpu as pltpu
from jax.experimental.pallas import tpu_sc as plsc

BATCH = 16384
DIM = 32

_info = plsc.get_sparse_core_info()
_NC, _NS, _L = _info.num_cores, _info.num_subcores, _info.num_lanes
_NW = _NC * _NS
_BPW = BATCH // _NW
_CHUNKS = _BPW // _L
_K = 8
_LOOK = 7


def _body(user_ids_hbm, item_ids_hbm, ut_hbm, it_hbm, out_hbm,
          uidx_v, iidx_v, tu_v, wu_v, ti_v, wi_v,
          u_ring, i_ring, out_v, sem_u, sem_i):
    wid = lax.axis_index("s") * _NC + lax.axis_index("c")
    base = wid * _BPW

    pltpu.sync_copy(user_ids_hbm.at[pl.ds(base, _BPW)], uidx_v)
    pltpu.sync_copy(item_ids_hbm.at[pl.ds(base, _BPW)], iidx_v)

    def pre_body(c, carry):
        sl = pl.ds(c * _L, _L)
        ju = uidx_v[sl]
        ji = iidx_v[sl]
        tu_v[sl] = lax.shift_right_logical(ju, 7)
        wu_v[sl] = lax.bitwise_and(ju, 127)
        ti_v[sl] = lax.shift_right_logical(ji, 7)
        wi_v[sl] = lax.bitwise_and(ji, 127)
        return carry

    lax.fori_loop(0, _CHUNKS, pre_body, 0)

    lane = lax.iota(jnp.int32, _L)
    lane_hi = lane + _L

    def fire(tu, ti, slot):
        pltpu.async_copy(
            ut_hbm.at[:, pl.ds(pl.multiple_of(tu * 128, 128), 128)],
            u_ring.at[:, pl.ds(slot * 128, 128)], sem_u.at[slot])
        pltpu.async_copy(
            it_hbm.at[:, pl.ds(pl.multiple_of(ti * 128, 128), 128)],
            i_ring.at[:, pl.ds(slot * 128, 128)], sem_i.at[slot])

    def drain(slot):
        pltpu.make_async_copy(
            ut_hbm.at[:, pl.ds(0, 128)],
            u_ring.at[:, pl.ds(slot * 128, 128)], sem_u.at[slot]).wait()
        pltpu.make_async_copy(
            it_hbm.at[:, pl.ds(0, 128)],
            i_ring.at[:, pl.ds(slot * 128, 128)], sem_i.at[slot]).wait()

    def extract(wu, wi, slot):
        cu = jnp.full((_L,), slot * 128 + wu, jnp.int32)
        ci = jnp.full((_L,), slot * 128 + wi, jnp.int32)
        u_lo = plsc.load_gather(u_ring, [lane, cu])
        u_hi = plsc.load_gather(u_ring, [lane_hi, cu])
        i_lo = plsc.load_gather(i_ring, [lane, ci])
        i_hi = plsc.load_gather(i_ring, [lane_hi, ci])
        return jnp.sum(u_lo * i_lo + u_hi * i_hi)

    tu0 = tu_v[pl.ds(0, _L)]
    ti0 = ti_v[pl.ds(0, _L)]
    for l in range(_LOOK):
        fire(tu0[l], ti0[l], l % _K)

    def chunk_body(c, carry):
        sl = pl.ds(c * _L, _L)
        wu_cur = wu_v[sl]
        wi_cur = wi_v[sl]
        tu_cur = tu_v[sl]
        ti_cur = ti_v[sl]
        nxt = pl.ds(jnp.minimum(c + 1, _CHUNKS - 1) * _L, _L)
        tu_nxt = tu_v[nxt]
        ti_nxt = ti_v[nxt]
        acc = jnp.zeros((_L,), jnp.float32)
        for l in range(_L):
            lf = l + _LOOK
            if lf < _L:
                fire(tu_cur[lf], ti_cur[lf], lf % _K)
            else:
                tun = tu_nxt[lf - _L]
                tin = ti_nxt[lf - _L]
                pl.when(c < _CHUNKS - 1)(
                    lambda tun=tun, tin=tin, lf=lf: fire(tun, tin, lf % _K))
            slot = l % _K
            drain(slot)
            s = extract(wu_cur[l], wi_cur[l], slot)
            acc = jnp.where(lane == l, s, acc)
        out_v[sl] = acc
        return carry

    lax.fori_loop(0, _CHUNKS, chunk_body, 0)

    pltpu.sync_copy(out_v, out_hbm.at[pl.ds(base, _BPW)])


@jax.jit
def _two_tower(user_ids, item_ids, user_emb_t, item_emb_t):
    mesh = plsc.VectorSubcoreMesh(core_axis_name="c", subcore_axis_name="s")
    kern = pl.kernel(
        _body,
        mesh=mesh,
        compiler_params=pltpu.CompilerParams(
            needs_layout_passes=False, use_tc_tiling_on_sc=True),
        out_type=jax.ShapeDtypeStruct((BATCH,), jnp.float32),
        scratch_types=[
            pltpu.VMEM((_BPW,), jnp.int32),
            pltpu.VMEM((_BPW,), jnp.int32),
            pltpu.VMEM((_BPW,), jnp.int32),
            pltpu.VMEM((_BPW,), jnp.int32),
            pltpu.VMEM((_BPW,), jnp.int32),
            pltpu.VMEM((_BPW,), jnp.int32),
            pltpu.VMEM((DIM, _K * 128), jnp.float32),
            pltpu.VMEM((DIM, _K * 128), jnp.float32),
            pltpu.VMEM((_BPW,), jnp.float32),
            pltpu.SemaphoreType.DMA((_K,)),
            pltpu.SemaphoreType.DMA((_K,)),
        ],
    )
    return kern(user_ids, item_ids, user_emb_t, item_emb_t)


def kernel(user_ids, item_ids, user_emb, item_emb):
    return _two_tower(user_ids.astype(jnp.int32), item_ids.astype(jnp.int32),
                      user_emb.T, item_emb.T)
